# baseline (speedup 1.0000x reference)
import functools

import jax
import jax.numpy as jnp
from jax import lax
from jax.experimental import pallas as pl
from jax.experimental.pallas import tpu as pltpu
from jax.experimental.pallas import tpu_sc as plsc

_NC = 2
_NS = 16
_NW = _NC * _NS
_C = 128
_EICH = 1024


def _build_sc_partition(half, ndeg, ept0, qcap):
    mesh = plsc.VectorSubcoreMesh(core_axis_name="c", subcore_axis_name="s")

    @functools.partial(
        pl.kernel,
        out_type=(jax.ShapeDtypeStruct((_NW * qcap,), jnp.int32),
                  jax.ShapeDtypeStruct((_NW * qcap,), jnp.int32),
                  jax.ShapeDtypeStruct((_NW * 128,), jnp.int32),
                  jax.ShapeDtypeStruct((_NW * ndeg,), jnp.float32)),
        mesh=mesh,
        scratch_types=(
            pltpu.VMEM((2, _EICH), jnp.int32),
            pltpu.VMEM((qcap,), jnp.int32),
            pltpu.VMEM((qcap,), jnp.int32),
            pltpu.VMEM((128,), jnp.int32),
            pltpu.VMEM((ndeg,), jnp.float32),
        ),
        compiler_params=pltpu.CompilerParams(needs_layout_passes=False))
    def part_kernel(edges_hbm, zq_hbm, zd_hbm,
                    qsrc_out, qdst_out, cnt_out, deg_out,
                    ei, qsrc, qdst, cnt, ldeg):
        cid = lax.axis_index("c")
        sid = lax.axis_index("s")
        w = cid * _NS + sid
        e0 = w * ept0
        lane = lax.iota(jnp.int32, 16)
        ones = jnp.ones((16,), jnp.float32)
        zero16 = jnp.zeros((16,), jnp.int32)

        pltpu.sync_copy(zq_hbm, qsrc)
        pltpu.sync_copy(zq_hbm, qdst)
        pltpu.sync_copy(zd_hbm, ldeg)

        def body(b, carry):
            q0, q1 = carry
            pltpu.sync_copy(edges_hbm.at[:, pl.ds(e0 + b * _EICH, _EICH)], ei)
            for j in range(_EICH // 16):
                s16 = ei[0, pl.ds(j * 16, 16)]
                d16 = ei[1, pl.ds(j * 16, 16)]
                plsc.addupdate_scatter(ldeg, [d16], ones)
                ok0 = d16 < half
                inc0 = jnp.where(ok0, 1, 0)
                c0 = plsc.cumsum(inc0)
                idx0 = q0 + c0 - 1
                plsc.store_scatter(qsrc, [idx0], s16, mask=ok0)
                plsc.store_scatter(qdst, [idx0], d16, mask=ok0)
                c1 = plsc.cumsum(1 - inc0)
                idx1 = (qcap - 1) - (q1 + c1 - 1)
                ok1 = jnp.logical_not(ok0)
                plsc.store_scatter(qsrc, [idx1], s16, mask=ok1)
                plsc.store_scatter(qdst, [idx1], d16 - half, mask=ok1)
                q0 = q0 + c0[15]
                q1 = q1 + c1[15]
            return (q0, q1)

        q0, q1 = lax.fori_loop(0, ept0 // _EICH, body, (0, 0))

        rem0 = (-q0) % 128
        rem1 = (-q1) % 128
        for j in range(8):
            pos = j * 16 + lane
            m0 = pos < rem0
            plsc.store_scatter(qsrc, [q0 + pos], zero16, mask=m0)
            plsc.store_scatter(qdst, [q0 + pos], half + lane, mask=m0)
            m1 = pos < rem1
            plsc.store_scatter(qsrc, [(qcap - 1) - (q1 + pos)], zero16,
                               mask=m1)
            plsc.store_scatter(qdst, [(qcap - 1) - (q1 + pos)], half + lane,
                               mask=m1)
        q0 = q0 + rem0
        q1 = q1 + rem1

        cvec = jnp.where(lane == 0, lax.shift_right_logical(q0, 7),
                         jnp.where(lane == 1, qcap - q1,
                                   jnp.where(lane == 2,
                                             lax.shift_right_logical(q1, 7),
                                             0)))
        cnt[pl.ds(0, 16)] = cvec
        for j in range(1, 8):
            cnt[pl.ds(j * 16, 16)] = zero16

        pltpu.sync_copy(qsrc, qsrc_out.at[pl.ds(w * qcap, qcap)])
        pltpu.sync_copy(qdst, qdst_out.at[pl.ds(w * qcap, qcap)])
        pltpu.sync_copy(cnt, cnt_out.at[pl.ds(w * 128, 128)])
        pltpu.sync_copy(ldeg, deg_out.at[pl.ds(w * ndeg, ndeg)])

    return part_kernel


def _build_sc_layer(half, qcap, h):
    nacc = half + 128
    mesh = plsc.VectorSubcoreMesh(core_axis_name="c", subcore_axis_name="s")

    @functools.partial(
        pl.kernel,
        out_type=jax.ShapeDtypeStruct((_NC * half, h), jnp.float32),
        mesh=mesh,
        scratch_types=(
            pltpu.VMEM((_C,), jnp.int32),
            pltpu.VMEM((_C,), jnp.int32),
            pltpu.VMEM((_C, h), jnp.float32),
            pltpu.SemaphoreType.DMA,
            pltpu.VMEM((_C,), jnp.int32),
            pltpu.VMEM((_C,), jnp.int32),
            pltpu.VMEM((_C, h), jnp.float32),
            pltpu.SemaphoreType.DMA,
            pltpu.VMEM((128,), jnp.int32),
            pltpu.VMEM((32, h), jnp.float32),
            pltpu.VMEM_SHARED((nacc, h), jnp.float32),
        ),
        compiler_params=pltpu.CompilerParams(needs_layout_passes=False))
    def layer_kernel(y_hbm, qsrc_hbm, qdst_hbm, cnt_hbm, zrow_hbm,
                     agg_out, si0, dl0, rw0, se0, si1, dl1, rw1, se1,
                     cnt, zstage, acc_sh):
        cid = lax.axis_index("c")
        sid = lax.axis_index("s")
        lo = cid * half

        pltpu.sync_copy(zrow_hbm, zstage)
        for kk in range(-(-(nacc // 32) // _NS)):
            sl = sid + kk * _NS

            @pl.when(sl < nacc // 32)
            def _():
                pltpu.sync_copy(zstage, acc_sh.at[pl.ds(sl * 32, 32)])
        plsc.subcore_barrier()

        slots = ((si0, dl0, rw0, se0), (si1, dl1, rw1, se1))

        def load_fire(slot, w, off):
            si, dl, rw, se = slot
            pltpu.sync_copy(qsrc_hbm.at[pl.ds(w * qcap + off, _C)], si)
            pltpu.sync_copy(qdst_hbm.at[pl.ds(w * qcap + off, _C)], dl)
            pltpu.async_copy(y_hbm.at[si], rw, se)

        def wait_gather(slot):
            si, dl, rw, se = slot
            pltpu.make_async_copy(y_hbm.at[si], rw, se).wait()

        def scatter(slot):
            si, dl, rw, se = slot
            pltpu.sync_copy(rw, acc_sh.at[dl], add=True)

        for t in range(2):
            w = 2 * sid + t
            pltpu.sync_copy(cnt_hbm.at[pl.ds(w * 128, 128)], cnt)
            cv = cnt[pl.ds(0, 16)]
            nch = jnp.where(cid == 0, cv[0], cv[2])
            qb = pl.multiple_of(jnp.where(cid == 0, 0, cv[1]), 128)

            @pl.when(nch > 0)
            def _():
                load_fire(slots[0], w, qb)

                def body(p, carry):
                    b = qb + 2 * p * _C
                    nxt1 = jnp.where(2 * p + 1 < nch, b + _C, qb)
                    load_fire(slots[1], w, nxt1)
                    wait_gather(slots[0])
                    scatter(slots[0])
                    nxt0 = jnp.where(2 * p + 2 < nch, b + 2 * _C, qb)
                    load_fire(slots[0], w, nxt0)
                    wait_gather(slots[1])

                    @pl.when(2 * p + 1 < nch)
                    def _():
                        scatter(slots[1])
                    return carry

                lax.fori_loop(0, (nch + 1) // 2, body, 0)
                wait_gather(slots[0])

        plsc.subcore_barrier()

        for kk in range(-(-(half // 128) // _NS)):
            sl = sid + kk * _NS

            @pl.when(sl < half // 128)
            def _():
                pltpu.sync_copy(acc_sh.at[pl.ds(sl * 128, 128)],
                                agg_out.at[pl.ds(lo + sl * 128, 128)])

    return layer_kernel


def _dot_t(a, w):
    return lax.dot_general(a, w, (((1,), (1,)), ((), ())),
                           preferred_element_type=jnp.float32)


def _lin_pair_body(x_ref, wl_ref, wr_ref, b_ref, yl_ref, yr_ref):
    x = x_ref[...]
    yl_ref[...] = _dot_t(x, wl_ref[...])
    yr_ref[...] = _dot_t(x, wr_ref[...]) + b_ref[...][None, :]


def _tc_lin_pair(x, wl, wr, b, bn):
    n, d = x.shape
    h = wl.shape[0]
    grid = pl.cdiv(n, bn)
    return pl.pallas_call(
        _lin_pair_body,
        grid=(grid,),
        in_specs=[
            pl.BlockSpec((bn, d), lambda i: (i, 0)),
            pl.BlockSpec((h, d), lambda i: (0, 0)),
            pl.BlockSpec((h, d), lambda i: (0, 0)),
            pl.BlockSpec((h,), lambda i: (0,)),
        ],
        out_specs=[
            pl.BlockSpec((bn, h), lambda i: (i, 0)),
            pl.BlockSpec((bn, h), lambda i: (i, 0)),
        ],
        out_shape=[
            jax.ShapeDtypeStruct((n, h), jnp.float32),
            jax.ShapeDtypeStruct((n, h), jnp.float32),
        ],
    )(x, wl, wr, b)


def _mid_body(agg_ref, deg_ref, r1_ref, wl_ref, wr_ref, b_ref,
              y2_ref, r2_ref, rdeg_ref):
    deg = jnp.sum(deg_ref[...], axis=0)
    rdeg = 1.0 / jnp.maximum(deg, 1.0)
    h = jnp.maximum(agg_ref[...] * rdeg[:, None] + r1_ref[...], 0.0)
    y2_ref[...] = _dot_t(h, wl_ref[...])
    r2_ref[...] = _dot_t(h, wr_ref[...]) + b_ref[...][None, :]
    rdeg_ref[...] = rdeg


def _tc_mid(agg, deg, r1, wl, wr, b, bn):
    n, h = r1.shape
    o = wl.shape[0]
    grid = pl.cdiv(n, bn)
    return pl.pallas_call(
        _mid_body,
        grid=(grid,),
        in_specs=[
            pl.BlockSpec((bn, h), lambda i: (i, 0)),
            pl.BlockSpec((_NW, bn), lambda i: (0, i)),
            pl.BlockSpec((bn, h), lambda i: (i, 0)),
            pl.BlockSpec((o, h), lambda i: (0, 0)),
            pl.BlockSpec((o, h), lambda i: (0, 0)),
            pl.BlockSpec((o,), lambda i: (0,)),
        ],
        out_specs=[
            pl.BlockSpec((bn, o), lambda i: (i, 0)),
            pl.BlockSpec((bn, o), lambda i: (i, 0)),
            pl.BlockSpec((bn,), lambda i: (i,)),
        ],
        out_shape=[
            jax.ShapeDtypeStruct((n, o), jnp.float32),
            jax.ShapeDtypeStruct((n, o), jnp.float32),
            jax.ShapeDtypeStruct((n,), jnp.float32),
        ],
    )(agg, deg, r1, wl, wr, b)


def _final_body(agg_ref, rdeg_ref, r2_ref, out_ref):
    out_ref[...] = agg_ref[...] * rdeg_ref[...][:, None] + r2_ref[...]


def _tc_final(agg, rdeg, r2, bn):
    n, o = r2.shape
    grid = pl.cdiv(n, bn)
    return pl.pallas_call(
        _final_body,
        grid=(grid,),
        in_specs=[
            pl.BlockSpec((bn, o), lambda i: (i, 0)),
            pl.BlockSpec((bn,), lambda i: (i,)),
            pl.BlockSpec((bn, o), lambda i: (i, 0)),
        ],
        out_specs=pl.BlockSpec((bn, o), lambda i: (i, 0)),
        out_shape=jax.ShapeDtypeStruct((n, o), jnp.float32),
    )(agg, rdeg, r2)


def kernel(x, edge_index, W1l, b1, W1r, W2l, b2, W2r):
    n, d = x.shape
    h = W1l.shape[0]

    e = edge_index.shape[1]
    ept0 = -(-e // (_NW * _EICH)) * _EICH
    e_pad = ept0 * _NW
    pad = e_pad - e
    edges = jnp.concatenate(
        [edge_index,
         jnp.stack([jnp.zeros((pad,), jnp.int32),
                    jnp.full((pad,), n, jnp.int32)])], axis=1)

    half = -(-n // (2 * 128)) * 128
    qcap = ept0 + 256
    ndeg = _NC * half

    part = _build_sc_partition(half, ndeg, ept0, qcap)
    scl = _build_sc_layer(half, qcap, h)
    zq = jnp.zeros((qcap,), jnp.int32)
    zd = jnp.zeros((ndeg,), jnp.float32)
    zrow = jnp.zeros((32, h), jnp.float32)

    bn = 512

    qsrc, qdst, cnt, dgp = part(edges, zq, zd)
    y1, r1 = _tc_lin_pair(x, W1l, W1r, b1, bn)
    agg1 = scl(y1, qsrc, qdst, cnt, zrow)
    y2, r2, rdeg = _tc_mid(agg1[:n], dgp.reshape(_NW, ndeg)[:, :n],
                           r1, W2l, W2r, b2, bn)
    agg2 = scl(y2, qsrc, qdst, cnt, zrow)
    return _tc_final(agg2[:n], rdeg, r2, bn)


# speedup vs baseline: 1.0054x; 1.0054x over previous
import functools

import jax
import jax.numpy as jnp
from jax import lax
from jax.experimental import pallas as pl
from jax.experimental.pallas import tpu as pltpu
from jax.experimental.pallas import tpu_sc as plsc

_NC = 2
_NS = 16
_NW = _NC * _NS
_C = 128
_EICH = 1024


def _build_sc_partition(half, ndeg, ept0, qcap):
    mesh = plsc.VectorSubcoreMesh(core_axis_name="c", subcore_axis_name="s")

    @functools.partial(
        pl.kernel,
        out_type=(jax.ShapeDtypeStruct((_NW * qcap,), jnp.int32),
                  jax.ShapeDtypeStruct((_NW * qcap,), jnp.int32),
                  jax.ShapeDtypeStruct((_NW * 128,), jnp.int32),
                  jax.ShapeDtypeStruct((_NW * ndeg,), jnp.float32)),
        mesh=mesh,
        scratch_types=(
            pltpu.VMEM((2, _EICH), jnp.int32),
            pltpu.VMEM((qcap,), jnp.int32),
            pltpu.VMEM((qcap,), jnp.int32),
            pltpu.VMEM((128,), jnp.int32),
            pltpu.VMEM((ndeg,), jnp.float32),
        ),
        compiler_params=pltpu.CompilerParams(needs_layout_passes=False))
    def part_kernel(edges_hbm, zq_hbm, zd_hbm,
                    qsrc_out, qdst_out, cnt_out, deg_out,
                    ei, qsrc, qdst, cnt, ldeg):
        cid = lax.axis_index("c")
        sid = lax.axis_index("s")
        w = cid * _NS + sid
        e0 = w * ept0
        lane = lax.iota(jnp.int32, 16)
        ones = jnp.ones((16,), jnp.float32)
        zero16 = jnp.zeros((16,), jnp.int32)

        pltpu.sync_copy(zq_hbm, qsrc)
        pltpu.sync_copy(zq_hbm, qdst)
        pltpu.sync_copy(zd_hbm, ldeg)

        def body(b, carry):
            q0, q1 = carry
            pltpu.sync_copy(edges_hbm.at[:, pl.ds(e0 + b * _EICH, _EICH)], ei)
            for j in range(_EICH // 16):
                s16 = ei[0, pl.ds(j * 16, 16)]
                d16 = ei[1, pl.ds(j * 16, 16)]
                plsc.addupdate_scatter(ldeg, [d16], ones)
                ok0 = d16 < half
                inc0 = jnp.where(ok0, 1, 0)
                c0 = plsc.cumsum(inc0)
                idx0 = q0 + c0 - 1
                plsc.store_scatter(qsrc, [idx0], s16, mask=ok0)
                plsc.store_scatter(qdst, [idx0], d16, mask=ok0)
                c1 = plsc.cumsum(1 - inc0)
                idx1 = (qcap - 1) - (q1 + c1 - 1)
                ok1 = jnp.logical_not(ok0)
                plsc.store_scatter(qsrc, [idx1], s16, mask=ok1)
                plsc.store_scatter(qdst, [idx1], d16 - half, mask=ok1)
                q0 = q0 + c0[15]
                q1 = q1 + c1[15]
            return (q0, q1)

        q0, q1 = lax.fori_loop(0, ept0 // _EICH, body, (0, 0))

        rem0 = (-q0) % 128
        rem1 = (-q1) % 128
        for j in range(8):
            pos = j * 16 + lane
            m0 = pos < rem0
            plsc.store_scatter(qsrc, [q0 + pos], zero16, mask=m0)
            plsc.store_scatter(qdst, [q0 + pos], half + pos, mask=m0)
            m1 = pos < rem1
            plsc.store_scatter(qsrc, [(qcap - 1) - (q1 + pos)], zero16,
                               mask=m1)
            plsc.store_scatter(qdst, [(qcap - 1) - (q1 + pos)], half + pos,
                               mask=m1)
        q0 = q0 + rem0
        q1 = q1 + rem1

        cvec = jnp.where(lane == 0, lax.shift_right_logical(q0, 7),
                         jnp.where(lane == 1, qcap - q1,
                                   jnp.where(lane == 2,
                                             lax.shift_right_logical(q1, 7),
                                             0)))
        cnt[pl.ds(0, 16)] = cvec
        for j in range(1, 8):
            cnt[pl.ds(j * 16, 16)] = zero16

        pltpu.sync_copy(qsrc, qsrc_out.at[pl.ds(w * qcap, qcap)])
        pltpu.sync_copy(qdst, qdst_out.at[pl.ds(w * qcap, qcap)])
        pltpu.sync_copy(cnt, cnt_out.at[pl.ds(w * 128, 128)])
        pltpu.sync_copy(ldeg, deg_out.at[pl.ds(w * ndeg, ndeg)])

    return part_kernel


def _build_sc_layer(half, qcap, h):
    nacc = half + 128
    mesh = plsc.VectorSubcoreMesh(core_axis_name="c", subcore_axis_name="s")

    @functools.partial(
        pl.kernel,
        out_type=jax.ShapeDtypeStruct((_NC * half, h), jnp.float32),
        mesh=mesh,
        scratch_types=(
            pltpu.VMEM((_C,), jnp.int32),
            pltpu.VMEM((_C,), jnp.int32),
            pltpu.VMEM((_C, h), jnp.float32),
            pltpu.SemaphoreType.DMA,
            pltpu.VMEM((_C,), jnp.int32),
            pltpu.VMEM((_C,), jnp.int32),
            pltpu.VMEM((_C, h), jnp.float32),
            pltpu.SemaphoreType.DMA,
            pltpu.VMEM((128,), jnp.int32),
            pltpu.VMEM((32, h), jnp.float32),
            pltpu.VMEM_SHARED((nacc, h), jnp.float32),
        ),
        compiler_params=pltpu.CompilerParams(needs_layout_passes=False))
    def layer_kernel(y_hbm, qsrc_hbm, qdst_hbm, cnt_hbm, zrow_hbm,
                     agg_out, si0, dl0, rw0, se0, si1, dl1, rw1, se1,
                     cnt, zstage, acc_sh):
        cid = lax.axis_index("c")
        sid = lax.axis_index("s")
        lo = cid * half

        pltpu.sync_copy(zrow_hbm, zstage)
        for kk in range(-(-(nacc // 32) // _NS)):
            sl = sid + kk * _NS

            @pl.when(sl < nacc // 32)
            def _():
                pltpu.sync_copy(zstage, acc_sh.at[pl.ds(sl * 32, 32)])
        plsc.subcore_barrier()

        slots = ((si0, dl0, rw0, se0), (si1, dl1, rw1, se1))

        def load_fire(slot, w, off):
            si, dl, rw, se = slot
            pltpu.sync_copy(qsrc_hbm.at[pl.ds(w * qcap + off, _C)], si)
            pltpu.sync_copy(qdst_hbm.at[pl.ds(w * qcap + off, _C)], dl)
            pltpu.async_copy(y_hbm.at[si], rw, se)

        def wait_gather(slot):
            si, dl, rw, se = slot
            pltpu.make_async_copy(y_hbm.at[si], rw, se).wait()

        def scatter(slot):
            si, dl, rw, se = slot
            pltpu.sync_copy(rw, acc_sh.at[dl], add=True)

        for t in range(2):
            w = 2 * sid + t
            pltpu.sync_copy(cnt_hbm.at[pl.ds(w * 128, 128)], cnt)
            cv = cnt[pl.ds(0, 16)]
            nch = jnp.where(cid == 0, cv[0], cv[2])
            qb = pl.multiple_of(jnp.where(cid == 0, 0, cv[1]), 128)

            @pl.when(nch > 0)
            def _():
                load_fire(slots[0], w, qb)

                def body(p, carry):
                    b = qb + 2 * p * _C
                    nxt1 = jnp.where(2 * p + 1 < nch, b + _C, qb)
                    load_fire(slots[1], w, nxt1)
                    wait_gather(slots[0])
                    scatter(slots[0])
                    nxt0 = jnp.where(2 * p + 2 < nch, b + 2 * _C, qb)
                    load_fire(slots[0], w, nxt0)
                    wait_gather(slots[1])

                    @pl.when(2 * p + 1 < nch)
                    def _():
                        scatter(slots[1])
                    return carry

                lax.fori_loop(0, (nch + 1) // 2, body, 0)
                wait_gather(slots[0])

        plsc.subcore_barrier()

        for kk in range(-(-(half // 128) // _NS)):
            sl = sid + kk * _NS

            @pl.when(sl < half // 128)
            def _():
                pltpu.sync_copy(acc_sh.at[pl.ds(sl * 128, 128)],
                                agg_out.at[pl.ds(lo + sl * 128, 128)])

    return layer_kernel


def _dot_t(a, w):
    return lax.dot_general(a, w, (((1,), (1,)), ((), ())),
                           preferred_element_type=jnp.float32)


def _lin_pair_body(x_ref, wl_ref, wr_ref, b_ref, yl_ref, yr_ref):
    x = x_ref[...]
    yl_ref[...] = _dot_t(x, wl_ref[...])
    yr_ref[...] = _dot_t(x, wr_ref[...]) + b_ref[...][None, :]


def _tc_lin_pair(x, wl, wr, b, bn):
    n, d = x.shape
    h = wl.shape[0]
    grid = pl.cdiv(n, bn)
    return pl.pallas_call(
        _lin_pair_body,
        grid=(grid,),
        in_specs=[
            pl.BlockSpec((bn, d), lambda i: (i, 0)),
            pl.BlockSpec((h, d), lambda i: (0, 0)),
            pl.BlockSpec((h, d), lambda i: (0, 0)),
            pl.BlockSpec((h,), lambda i: (0,)),
        ],
        out_specs=[
            pl.BlockSpec((bn, h), lambda i: (i, 0)),
            pl.BlockSpec((bn, h), lambda i: (i, 0)),
        ],
        out_shape=[
            jax.ShapeDtypeStruct((n, h), jnp.float32),
            jax.ShapeDtypeStruct((n, h), jnp.float32),
        ],
    )(x, wl, wr, b)


def _mid_body(agg_ref, deg_ref, r1_ref, wl_ref, wr_ref, b_ref,
              y2_ref, r2_ref, rdeg_ref):
    deg = jnp.sum(deg_ref[...], axis=0)
    rdeg = 1.0 / jnp.maximum(deg, 1.0)
    h = jnp.maximum(agg_ref[...] * rdeg[:, None] + r1_ref[...], 0.0)
    y2_ref[...] = _dot_t(h, wl_ref[...])
    r2_ref[...] = _dot_t(h, wr_ref[...]) + b_ref[...][None, :]
    rdeg_ref[...] = rdeg


def _tc_mid(agg, deg, r1, wl, wr, b, bn):
    n, h = r1.shape
    o = wl.shape[0]
    grid = pl.cdiv(n, bn)
    return pl.pallas_call(
        _mid_body,
        grid=(grid,),
        in_specs=[
            pl.BlockSpec((bn, h), lambda i: (i, 0)),
            pl.BlockSpec((_NW, bn), lambda i: (0, i)),
            pl.BlockSpec((bn, h), lambda i: (i, 0)),
            pl.BlockSpec((o, h), lambda i: (0, 0)),
            pl.BlockSpec((o, h), lambda i: (0, 0)),
            pl.BlockSpec((o,), lambda i: (0,)),
        ],
        out_specs=[
            pl.BlockSpec((bn, o), lambda i: (i, 0)),
            pl.BlockSpec((bn, o), lambda i: (i, 0)),
            pl.BlockSpec((bn,), lambda i: (i,)),
        ],
        out_shape=[
            jax.ShapeDtypeStruct((n, o), jnp.float32),
            jax.ShapeDtypeStruct((n, o), jnp.float32),
            jax.ShapeDtypeStruct((n,), jnp.float32),
        ],
    )(agg, deg, r1, wl, wr, b)


def _final_body(agg_ref, rdeg_ref, r2_ref, out_ref):
    out_ref[...] = agg_ref[...] * rdeg_ref[...][:, None] + r2_ref[...]


def _tc_final(agg, rdeg, r2, bn):
    n, o = r2.shape
    grid = pl.cdiv(n, bn)
    return pl.pallas_call(
        _final_body,
        grid=(grid,),
        in_specs=[
            pl.BlockSpec((bn, o), lambda i: (i, 0)),
            pl.BlockSpec((bn,), lambda i: (i,)),
            pl.BlockSpec((bn, o), lambda i: (i, 0)),
        ],
        out_specs=pl.BlockSpec((bn, o), lambda i: (i, 0)),
        out_shape=jax.ShapeDtypeStruct((n, o), jnp.float32),
    )(agg, rdeg, r2)


def kernel(x, edge_index, W1l, b1, W1r, W2l, b2, W2r):
    n, d = x.shape
    h = W1l.shape[0]

    e = edge_index.shape[1]
    ept0 = -(-e // (_NW * _EICH)) * _EICH
    e_pad = ept0 * _NW
    pad = e_pad - e
    half = -(-n // (2 * 128)) * 128
    pad_dst = n + jnp.arange(pad, dtype=jnp.int32) % (2 * half - n)
    edges = jnp.concatenate(
        [edge_index,
         jnp.stack([jnp.zeros((pad,), jnp.int32), pad_dst])], axis=1)

    qcap = ept0 + 256
    ndeg = _NC * half

    part = _build_sc_partition(half, ndeg, ept0, qcap)
    scl = _build_sc_layer(half, qcap, h)
    zq = jnp.zeros((qcap,), jnp.int32)
    zd = jnp.zeros((ndeg,), jnp.float32)
    zrow = jnp.zeros((32, h), jnp.float32)

    bn = 512

    qsrc, qdst, cnt, dgp = part(edges, zq, zd)
    y1, r1 = _tc_lin_pair(x, W1l, W1r, b1, bn)
    agg1 = scl(y1, qsrc, qdst, cnt, zrow)
    y2, r2, rdeg = _tc_mid(agg1[:n], dgp.reshape(_NW, ndeg)[:, :n],
                           r1, W2l, W2r, b2, bn)
    agg2 = scl(y2, qsrc, qdst, cnt, zrow)
    return _tc_final(agg2[:n], rdeg, r2, bn)


# speedup vs baseline: 1.5453x; 1.5370x over previous
import functools

import jax
import jax.numpy as jnp
from jax import lax
from jax.experimental import pallas as pl
from jax.experimental.pallas import tpu as pltpu
from jax.experimental.pallas import tpu_sc as plsc

_NC = 2
_NS = 16
_C = 128


def _build_sc_segsum(half, nloc, ept, h, with_deg):
    n_chunks = ept // _C
    assert n_chunks % 2 == 0
    n_pairs = n_chunks // 2
    rptl = nloc // _NS
    assert half % 128 == 0
    nacc = half + 128
    mesh = plsc.VectorSubcoreMesh(core_axis_name="c", subcore_axis_name="s")

    out_type = [jax.ShapeDtypeStruct((_NC * half, h), jnp.float32)]
    scratch = [
        pltpu.VMEM((2, _C), jnp.int32),
        pltpu.VMEM((_C,), jnp.int32),
        pltpu.VMEM((_C, h), jnp.float32),
        pltpu.SemaphoreType.DMA,
        pltpu.VMEM((2, _C), jnp.int32),
        pltpu.VMEM((_C,), jnp.int32),
        pltpu.VMEM((_C, h), jnp.float32),
        pltpu.SemaphoreType.DMA,
        pltpu.VMEM((32, h), jnp.float32),
        pltpu.VMEM_SHARED((nacc, h), jnp.float32),
    ]
    if with_deg:
        out_type.append(jax.ShapeDtypeStruct((_NC, _NS, nloc), jnp.float32))
        scratch += [
            pltpu.VMEM((nloc,), jnp.float32),
        ]

    @functools.partial(pl.kernel,
                       out_type=tuple(out_type) if with_deg else out_type[0],
                       mesh=mesh, scratch_types=tuple(scratch),
                       compiler_params=pltpu.CompilerParams(
                           needs_layout_passes=False))
    def sc_kernel(y_hbm, edges_hbm, zrow_hbm, zcol_hbm, *rest):
        if with_deg:
            (agg_out, deg_out, ei0, dl0, rw0, se0, ei1, dl1, rw1, se1,
             zstage, acc_sh, ldeg) = rest
        else:
            (agg_out, ei0, dl0, rw0, se0, ei1, dl1, rw1, se1,
             zstage, acc_sh) = rest
        slot0 = (ei0, dl0, rw0, se0)
        slot1 = (ei1, dl1, rw1, se1)
        cid = lax.axis_index("c")
        sid = lax.axis_index("s")
        lo = cid * half

        r0 = sid * rptl
        pltpu.sync_copy(zrow_hbm, zstage)
        for kk in range(-(-(nacc // 32) // _NS)):
            sl = sid + kk * _NS

            @pl.when(sl < nacc // 32)
            def _():
                pltpu.sync_copy(zstage, acc_sh.at[pl.ds(sl * 32, 32)])
        if with_deg:
            pltpu.sync_copy(zcol_hbm, ldeg)
        plsc.subcore_barrier()

        e0 = sid * ept
        lane = lax.iota(jnp.int32, 16)

        def load_fire(slot, base):
            ei, dl, rw, se = slot
            pltpu.sync_copy(edges_hbm.at[:, pl.ds(base, _C)], ei)
            pltpu.async_copy(y_hbm.at[ei.at[0]], rw, se)

        def remap(slot):
            ei, dl, rw, se = slot
            ones = jnp.ones((16,), jnp.float32)
            for j in range(_C // 16):
                d16 = ei[1, pl.ds(j * 16, 16)] - lo
                ok = (d16 >= 0) & (d16 < half)
                d16 = jnp.where(ok, d16, half + j * 16 + lane)
                dl[pl.ds(j * 16, 16)] = d16
                if with_deg:
                    plsc.addupdate_scatter(ldeg, [d16], ones)

        def drain_scatter(slot):
            ei, dl, rw, se = slot
            pltpu.make_async_copy(y_hbm.at[ei.at[0]], rw, se).wait()
            pltpu.sync_copy(rw, acc_sh.at[dl], add=True)

        load_fire(slot0, e0)

        def body(p, carry):
            b = e0 + 2 * p * _C
            load_fire(slot1, b + _C)
            remap(slot0)
            drain_scatter(slot0)
            load_fire(slot0, b + 2 * _C)
            remap(slot1)
            drain_scatter(slot1)
            return carry

        lax.fori_loop(0, n_pairs, body, 0)
        ei0, dl0, rw0, se0 = slot0
        pltpu.make_async_copy(y_hbm.at[ei0.at[0]], rw0, se0).wait()

        if with_deg:
            pltpu.sync_copy(ldeg, deg_out.at[cid, sid])

        plsc.subcore_barrier()

        for kk in range(-(-(half // 128) // _NS)):
            sl = sid + kk * _NS

            @pl.when(sl < half // 128)
            def _():
                pltpu.sync_copy(acc_sh.at[pl.ds(sl * 128, 128)],
                                agg_out.at[pl.ds(lo + sl * 128, 128)])

    return sc_kernel


def _dot_t(a, w):
    return lax.dot_general(a, w, (((1,), (1,)), ((), ())),
                           preferred_element_type=jnp.float32)


def _lin_pair_body(x_ref, wl_ref, wr_ref, b_ref, yl_ref, yr_ref):
    x = x_ref[...]
    yl_ref[...] = _dot_t(x, wl_ref[...])
    yr_ref[...] = _dot_t(x, wr_ref[...]) + b_ref[...][None, :]


def _tc_lin_pair(x, wl, wr, b, bn):
    n, d = x.shape
    h = wl.shape[0]
    grid = pl.cdiv(n, bn)
    return pl.pallas_call(
        _lin_pair_body,
        grid=(grid,),
        in_specs=[
            pl.BlockSpec((bn, d), lambda i: (i, 0)),
            pl.BlockSpec((h, d), lambda i: (0, 0)),
            pl.BlockSpec((h, d), lambda i: (0, 0)),
            pl.BlockSpec((h,), lambda i: (0,)),
        ],
        out_specs=[
            pl.BlockSpec((bn, h), lambda i: (i, 0)),
            pl.BlockSpec((bn, h), lambda i: (i, 0)),
        ],
        out_shape=[
            jax.ShapeDtypeStruct((n, h), jnp.float32),
            jax.ShapeDtypeStruct((n, h), jnp.float32),
        ],
    )(x, wl, wr, b)


def _mid_body(agg_ref, deg_ref, r1_ref, wl_ref, wr_ref, b_ref,
              y2_ref, r2_ref, rdeg_ref):
    deg = jnp.sum(deg_ref[0], axis=0)
    rdeg = 1.0 / jnp.maximum(deg, 1.0)
    h = jnp.maximum(agg_ref[...] * rdeg[:, None] + r1_ref[...], 0.0)
    y2_ref[...] = _dot_t(h, wl_ref[...])
    r2_ref[...] = _dot_t(h, wr_ref[...]) + b_ref[...][None, :]
    rdeg_ref[...] = rdeg


def _tc_mid(agg, deg, r1, wl, wr, b, bn):
    n, h = r1.shape
    o = wl.shape[0]
    half = deg.shape[2]
    assert half % bn == 0
    nbh = half // bn
    grid = pl.cdiv(n, bn)
    return pl.pallas_call(
        _mid_body,
        grid=(grid,),
        in_specs=[
            pl.BlockSpec((bn, h), lambda i: (i, 0)),
            pl.BlockSpec((1, _NS, bn), lambda i: (i // nbh, 0, i % nbh)),
            pl.BlockSpec((bn, h), lambda i: (i, 0)),
            pl.BlockSpec((o, h), lambda i: (0, 0)),
            pl.BlockSpec((o, h), lambda i: (0, 0)),
            pl.BlockSpec((o,), lambda i: (0,)),
        ],
        out_specs=[
            pl.BlockSpec((bn, o), lambda i: (i, 0)),
            pl.BlockSpec((bn, o), lambda i: (i, 0)),
            pl.BlockSpec((bn,), lambda i: (i,)),
        ],
        out_shape=[
            jax.ShapeDtypeStruct((n, o), jnp.float32),
            jax.ShapeDtypeStruct((n, o), jnp.float32),
            jax.ShapeDtypeStruct((n,), jnp.float32),
        ],
    )(agg, deg, r1, wl, wr, b)


def _final_body(agg_ref, rdeg_ref, r2_ref, out_ref):
    out_ref[...] = agg_ref[...] * rdeg_ref[...][:, None] + r2_ref[...]


def _tc_final(agg, rdeg, r2, bn):
    n, o = r2.shape
    grid = pl.cdiv(n, bn)
    return pl.pallas_call(
        _final_body,
        grid=(grid,),
        in_specs=[
            pl.BlockSpec((bn, o), lambda i: (i, 0)),
            pl.BlockSpec((bn,), lambda i: (i,)),
            pl.BlockSpec((bn, o), lambda i: (i, 0)),
        ],
        out_specs=pl.BlockSpec((bn, o), lambda i: (i, 0)),
        out_shape=jax.ShapeDtypeStruct((n, o), jnp.float32),
    )(agg, rdeg, r2)


def kernel(x, edge_index, W1l, b1, W1r, W2l, b2, W2r):
    n, d = x.shape
    h = W1l.shape[0]

    e = edge_index.shape[1]
    ept = -(-e // (_NS * 2 * _C)) * 2 * _C
    e_pad = ept * _NS + _C
    pad = e_pad - e
    edges = jnp.concatenate(
        [edge_index,
         jnp.stack([jnp.zeros((pad,), jnp.int32),
                    jnp.full((pad,), n, jnp.int32)])], axis=1)

    half = -(-n // (2 * 128)) * 128
    nloc = -(-(half + 1) // 2048) * 2048

    sc1 = _build_sc_segsum(half, nloc, ept, h, with_deg=True)
    sc2 = _build_sc_segsum(half, nloc, ept, h, with_deg=False)
    zrow = jnp.zeros((32, h), jnp.float32)
    zcol = jnp.zeros((nloc,), jnp.float32)

    bn = 512

    y1, r1 = _tc_lin_pair(x, W1l, W1r, b1, bn)
    agg1, dgp = sc1(y1, edges, zrow, zcol)
    agg1 = agg1[:n]
    y2, r2, rdeg = _tc_mid(agg1, dgp[:, :, :half], r1, W2l, W2r, b2, bn)
    agg2 = sc2(y2, edges, zrow, zcol)
    agg2 = agg2[:n]
    return _tc_final(agg2, rdeg, r2, bn)
